# baseline (speedup 1.0000x reference)
import functools

import jax
import jax.numpy as jnp
from jax import lax
from jax.experimental import pallas as pl
from jax.experimental.pallas import tpu as pltpu
from jax.experimental.pallas import tpu_sc as plsc

NUM_EXPERTS = 8
TOP_K = 2
BLK = 4096
L = 16


def _matmul_block(x_ref, w_ref, logits_t_ref):
    logits_t_ref[...] = jax.lax.dot_general(
        w_ref[...], x_ref[...], (((1,), (1,)), ((), ())),
        preferred_element_type=jnp.float32,
    )


def _make_sc_topk(n):
    info = plsc.get_sparse_core_info()
    nc, ns = info.num_cores, info.num_subcores
    nw = nc * ns
    chunk = n // nw
    mesh = plsc.VectorSubcoreMesh(core_axis_name="c", subcore_axis_name="s")

    @functools.partial(
        pl.kernel,
        mesh=mesh,
        out_type=jax.ShapeDtypeStruct((4, n), jnp.float32),
        scratch_types=[
            pltpu.VMEM((NUM_EXPERTS, chunk), jnp.float32),
            pltpu.VMEM((4, chunk), jnp.float32),
        ],
    )
    def sc_topk(logits_hbm, aux_hbm, logits_v, aux_v):
        wid = lax.axis_index("s") * nc + lax.axis_index("c")
        base = wid * chunk
        pltpu.sync_copy(logits_hbm.at[:, pl.ds(base, chunk)], logits_v)

        def body(g, _):
            t = g * L
            m1 = logits_v[0, pl.ds(t, L)]
            i1 = jnp.zeros((L,), jnp.float32)
            m2 = jnp.full((L,), -jnp.inf, jnp.float32)
            i2 = jnp.zeros((L,), jnp.float32)
            for e in range(1, NUM_EXPERTS):
                le = logits_v[e, pl.ds(t, L)]
                ef = jnp.full((L,), float(e), jnp.float32)
                beats1 = le > m1
                beats2 = le > m2
                m2 = jnp.where(beats1, m1, jnp.where(beats2, le, m2))
                i2 = jnp.where(beats1, i1, jnp.where(beats2, ef, i2))
                m1 = jnp.where(beats1, le, m1)
                i1 = jnp.where(beats1, ef, i1)
            ex = jnp.exp(m2 - m1)
            w2 = ex / (1.0 + ex)
            aux_v[0, pl.ds(t, L)] = i1
            aux_v[1, pl.ds(t, L)] = i2
            aux_v[2, pl.ds(t, L)] = 1.0 - w2
            aux_v[3, pl.ds(t, L)] = w2
            return _

        lax.fori_loop(0, chunk // L, body, 0)
        pltpu.sync_copy(aux_v, aux_hbm.at[:, pl.ds(base, chunk)])

    return sc_topk


@jax.jit
def kernel(hidden_states, W_gate):
    b, s, d = hidden_states.shape
    n = b * s
    x = hidden_states.reshape(n, d)

    nchunk = 4
    cn = n // nchunk
    sc_topk = _make_sc_topk(cn)
    logit_chunks = []
    aux_chunks = []
    for c in range(nchunk):
        lt_c = pl.pallas_call(
            _matmul_block,
            grid=(cn // BLK,),
            in_specs=[
                pl.BlockSpec((BLK, d), lambda i, c=c: (c * (cn // BLK) + i, 0)),
                pl.BlockSpec((NUM_EXPERTS, d), lambda i: (0, 0)),
            ],
            out_specs=pl.BlockSpec((NUM_EXPERTS, BLK), lambda i: (0, i)),
            out_shape=jax.ShapeDtypeStruct((NUM_EXPERTS, cn), jnp.float32),
        )(x, W_gate)
        logit_chunks.append(lt_c)
        aux_chunks.append(sc_topk(lt_c))

    logits_t = jnp.concatenate(logit_chunks, axis=1)
    aux = jnp.concatenate(aux_chunks, axis=1)
    router_logits = logits_t.T
    topk_idx = aux[0:TOP_K].T.astype(jnp.int32)
    expert_weights = aux[TOP_K : 2 * TOP_K].T
    return (router_logits, topk_idx, expert_weights)


# speedup vs baseline: 1.7945x; 1.7945x over previous
import jax
import jax.numpy as jnp
from jax.experimental import pallas as pl

NUM_EXPERTS = 8
TOP_K = 2
BLK = 4096


def _router_block(x_ref, w_ref, logits_t_ref, aux_ref):
    x = x_ref[...]
    w = w_ref[...]
    logits_t = jax.lax.dot_general(
        w, x, (((1,), (1,)), ((), ())), preferred_element_type=jnp.float32
    )
    logits_t_ref[...] = logits_t

    eidx = jax.lax.broadcasted_iota(jnp.int32, logits_t.shape, 0)
    big = jnp.int32(NUM_EXPERTS)

    m1 = jnp.max(logits_t, axis=0, keepdims=True)
    i1 = jnp.min(jnp.where(logits_t == m1, eidx, big), axis=0, keepdims=True)
    masked = jnp.where(eidx == i1, -jnp.inf, logits_t)
    m2 = jnp.max(masked, axis=0, keepdims=True)
    i2 = jnp.min(jnp.where(masked == m2, eidx, big), axis=0, keepdims=True)

    e = jnp.exp(m2 - m1)
    w2 = e / (1.0 + e)
    w1 = 1.0 - w2
    aux_ref[...] = jnp.concatenate(
        [i1.astype(jnp.float32), i2.astype(jnp.float32), w1, w2], axis=0
    )


@jax.jit
def kernel(hidden_states, W_gate):
    b, s, d = hidden_states.shape
    n = b * s
    x = hidden_states.reshape(n, d)

    grid = (n // BLK,)
    out_shapes = (
        jax.ShapeDtypeStruct((NUM_EXPERTS, n), jnp.float32),
        jax.ShapeDtypeStruct((4, n), jnp.float32),
    )
    logits_t, aux = pl.pallas_call(
        _router_block,
        grid=grid,
        in_specs=[
            pl.BlockSpec((BLK, d), lambda i: (i, 0)),
            pl.BlockSpec((NUM_EXPERTS, d), lambda i: (0, 0)),
        ],
        out_specs=(
            pl.BlockSpec((NUM_EXPERTS, BLK), lambda i: (0, i)),
            pl.BlockSpec((4, BLK), lambda i: (0, i)),
        ),
        out_shape=out_shapes,
    )(x, W_gate)

    router_logits = logits_t.T
    topk_idx = aux[0:TOP_K].T.astype(jnp.int32)
    expert_weights = aux[TOP_K : 2 * TOP_K].T
    return (router_logits, topk_idx, expert_weights)
